# baseline (speedup 1.0000x reference)
import functools

import jax
import jax.numpy as jnp
from jax import lax
from jax.experimental import pallas as pl
from jax.experimental.pallas import tpu as pltpu
from jax.experimental.pallas import tpu_sc as plsc

N = 10000
NP = 10112
DI = 128
DE = 16
DH = 256
E = 320000
K = 64
CH = 158
EPT = CH * K
NT = 32
EPAD = NT * EPT
RPT = NP // 16
EB = 2048

_MESH = plsc.VectorSubcoreMesh(core_axis_name="c", subcore_axis_name="s")


def _emb_body(ea_ref, we_ref, be_ref, o_ref):
    o_ref[...] = (jnp.dot(ea_ref[...], we_ref[...],
                          preferred_element_type=jnp.float32) + be_ref[...])


def _edge_emb(ea_pad, W_e, b_e2d):
    return pl.pallas_call(
        _emb_body,
        grid=(EPAD // EB,),
        in_specs=[
            pl.BlockSpec((EB, DE), lambda i: (i, 0)),
            pl.BlockSpec((DE, DI), lambda i: (0, 0)),
            pl.BlockSpec((1, DI), lambda i: (0, 0)),
        ],
        out_specs=pl.BlockSpec((EB, DI), lambda i: (i, 0)),
        out_shape=jax.ShapeDtypeStruct((EPAD, DI), jnp.float32),
    )(ea_pad, W_e, b_e2d)


def _sc_aggregate():
    @functools.partial(
        pl.kernel,
        mesh=_MESH,
        out_type=[jax.ShapeDtypeStruct((2, NP, DI), jnp.float32)],
        scratch_types=[
            pltpu.VMEM((2, 2, K), jnp.int32),
            pltpu.VMEM((2, K, DI), jnp.float32),
            pltpu.VMEM((2, K, DI), jnp.float32),
            pltpu.VMEM_SHARED((NP, DI), jnp.float32),
            pltpu.SemaphoreType.DMA,
            pltpu.SemaphoreType.DMA,
            pltpu.SemaphoreType.DMA,
            pltpu.SemaphoreType.DMA,
            pltpu.SemaphoreType.DMA,
            pltpu.SemaphoreType.DMA,
        ],
    )
    def k(x_hbm, sd_hbm, emb_hbm, agg_out,
          idx_v, rows_v, emb_v, acc, sg0, sg1, se0, se1, si0, si1):
        cid = lax.axis_index("c")
        sid = lax.axis_index("s")
        wid = cid * 16 + sid
        r0 = sid * RPT
        ebase = wid * EPT
        sg = (sg0, sg1)
        se = (se0, se1)
        si = (si0, si1)

        def zrow(q, c2):
            for c in range(DI // 16):
                rows_v[0, q, pl.ds(c * 16, 16)] = jnp.zeros((16,), jnp.float32)
            return c2

        lax.fori_loop(0, K, zrow, 0)
        nz = RPT // K + 1
        for t in range(nz):
            off = t * K
            nrows = K if t < nz - 1 else RPT - off
            pltpu.sync_copy(rows_v.at[0, pl.ds(0, nrows)],
                            acc.at[pl.ds(r0 + off, nrows)])
        plsc.subcore_barrier()

        def start_idx(j, p):
            pltpu.async_copy(sd_hbm.at[wid, j], idx_v.at[p], si[p])

        def wait_idx(p):
            pltpu.make_async_copy(sd_hbm.at[wid, 0], idx_v.at[p], si[p]).wait()

        def start_ge(j, p):
            pltpu.async_copy(x_hbm.at[idx_v.at[p, 0]], rows_v.at[p], sg[p])
            pltpu.async_copy(emb_hbm.at[pl.ds(ebase + j * K, K)],
                             emb_v.at[p], se[p])

        def wait_ge(p):
            pltpu.make_async_copy(x_hbm.at[pl.ds(0, K)], rows_v.at[p], sg[p]).wait()
            pltpu.make_async_copy(emb_hbm.at[pl.ds(0, K)], emb_v.at[p], se[p]).wait()

        start_idx(0, 0)
        wait_idx(0)
        start_ge(0, 0)
        start_idx(1, 1)

        def body(b, carry):
            for p in range(2):
                j = 2 * b + p
                wait_ge(p)
                pltpu.sync_copy(emb_v.at[p], acc.at[idx_v.at[p, 1]], add=True)
                pltpu.sync_copy(rows_v.at[p], acc.at[idx_v.at[p, 1]], add=True)

                @pl.when(j + 2 < CH)
                def _():
                    start_idx(j + 2, p)

                @pl.when(j + 1 < CH)
                def _():
                    wait_idx(1 - p)
                    start_ge(j + 1, 1 - p)

            return carry

        lax.fori_loop(0, CH // 2, body, 0)
        plsc.subcore_barrier()

        for t in range(nz):
            off = t * K
            nrows = K if t < nz - 1 else RPT - off
            pltpu.sync_copy(acc.at[pl.ds(r0 + off, nrows)],
                            rows_v.at[0, pl.ds(0, nrows)])
            pltpu.sync_copy(rows_v.at[0, pl.ds(0, nrows)],
                            agg_out.at[cid, pl.ds(r0 + off, nrows)])

    return k


def _tc_body(x_ref, agg_ref, w1_ref, b1_ref,
             g1_ref, bt1_ref, w2_ref, b2_ref, g2_ref, bt2_ref, eps_ref, o_ref):
    agg = agg_ref[0, :N, :] + agg_ref[1, :N, :]
    xx = x_ref[...]
    h = (1.0 + eps_ref[0, 0]) * xx + agg
    z = jnp.dot(h, w1_ref[...], preferred_element_type=jnp.float32) + b1_ref[...]
    m = jnp.mean(z, axis=0, keepdims=True)
    v = jnp.mean((z - m) ** 2, axis=0, keepdims=True)
    z = jnp.maximum((z - m) * lax.rsqrt(v + 1e-5) * g1_ref[...] + bt1_ref[...], 0.0)
    z2 = jnp.dot(z, w2_ref[...], preferred_element_type=jnp.float32) + b2_ref[...]
    m2 = jnp.mean(z2, axis=0, keepdims=True)
    v2 = jnp.mean((z2 - m2) ** 2, axis=0, keepdims=True)
    o_ref[...] = jnp.maximum((z2 - m2) * lax.rsqrt(v2 + 1e-5) * g2_ref[...] + bt2_ref[...], 0.0)


def kernel(x, edge_index, edge_attr, W_e, b_e, W1, b1, g1, beta1, W2, b2, g2, beta2, eps):
    src = edge_index[0].astype(jnp.int32)
    dst = edge_index[1].astype(jnp.int32)
    pad = EPAD - E
    src3d = jnp.concatenate([src, jnp.zeros((pad,), jnp.int32)]).reshape(NT, CH, K)
    dst_pad = N + (jnp.arange(pad, dtype=jnp.int32) % (NP - N))
    dst3d = jnp.concatenate([dst, dst_pad]).reshape(NT, CH, K)
    srcdst = jnp.stack([src3d, dst3d], axis=2)
    ea_pad = jnp.concatenate([edge_attr, jnp.zeros((pad, DE), edge_attr.dtype)])

    emb = _edge_emb(ea_pad, W_e, b_e.reshape(1, DI))
    (agg_pk,) = _sc_aggregate()(x, srcdst, emb)

    out = pl.pallas_call(
        _tc_body,
        out_shape=jax.ShapeDtypeStruct((N, DI), jnp.float32),
    )(x, agg_pk, W1, b1.reshape(1, DH),
      g1.reshape(1, DH), beta1.reshape(1, DH), W2, b2.reshape(1, DI),
      g2.reshape(1, DI), beta2.reshape(1, DI), eps.reshape(1, 1))
    return out


# speedup vs baseline: 1.2464x; 1.2464x over previous
import functools

import jax
import jax.numpy as jnp
from jax import lax
from jax.experimental import pallas as pl
from jax.experimental.pallas import tpu as pltpu
from jax.experimental.pallas import tpu_sc as plsc

N = 10000
NP = 10112
DI = 128
DE = 16
DH = 256
E = 320000
K = 64
CH = 158
EPT = CH * K
NT = 32
EPAD = NT * EPT
RPT = NP // 16
EB = 4096

_MESH = plsc.VectorSubcoreMesh(core_axis_name="c", subcore_axis_name="s")


def _emb_body(ea_ref, we_ref, be_ref, o_ref):
    o_ref[...] = (jnp.dot(ea_ref[...], we_ref[...],
                          preferred_element_type=jnp.float32) + be_ref[...])


def _edge_emb(ea_pad, W_e, b_e2d):
    return pl.pallas_call(
        _emb_body,
        grid=(EPAD // EB,),
        in_specs=[
            pl.BlockSpec((EB, DE), lambda i: (i, 0)),
            pl.BlockSpec((DE, DI), lambda i: (0, 0)),
            pl.BlockSpec((1, DI), lambda i: (0, 0)),
        ],
        out_specs=pl.BlockSpec((EB, DI), lambda i: (i, 0)),
        out_shape=jax.ShapeDtypeStruct((EPAD, DI), jnp.float32),
    )(ea_pad, W_e, b_e2d)


def _sc_aggregate():
    @functools.partial(
        pl.kernel,
        mesh=_MESH,
        out_type=[jax.ShapeDtypeStruct((2, NP, DI), jnp.float32)],
        scratch_types=[
            pltpu.VMEM((2, 2, K), jnp.int32),
            pltpu.VMEM((2, K, DI), jnp.float32),
            pltpu.VMEM((2, K, DI), jnp.float32),
            pltpu.VMEM_SHARED((NP, DI), jnp.float32),
            pltpu.SemaphoreType.DMA,
            pltpu.SemaphoreType.DMA,
            pltpu.SemaphoreType.DMA,
            pltpu.SemaphoreType.DMA,
        ],
    )
    def k(x_hbm, sd_hbm, emb_hbm, agg_out,
          idx_v, rows_v, emb_v, acc, sg0, sg1, se0, se1):
        cid = lax.axis_index("c")
        sid = lax.axis_index("s")
        wid = cid * 16 + sid
        r0 = sid * RPT
        ebase = wid * EPT
        sg = (sg0, sg1)
        se = (se0, se1)

        def zrow(q, c2):
            for c in range(DI // 16):
                rows_v[0, q, pl.ds(c * 16, 16)] = jnp.zeros((16,), jnp.float32)
            return c2

        lax.fori_loop(0, K, zrow, 0)
        nz = RPT // K + 1
        for t in range(nz):
            off = t * K
            nrows = K if t < nz - 1 else RPT - off
            pltpu.sync_copy(rows_v.at[0, pl.ds(0, nrows)],
                            acc.at[pl.ds(r0 + off, nrows)])
        plsc.subcore_barrier()

        def start_chunk(j, p):
            pltpu.sync_copy(sd_hbm.at[wid, j], idx_v.at[p])
            pltpu.async_copy(x_hbm.at[idx_v.at[p, 0]], rows_v.at[p], sg[p])
            pltpu.async_copy(emb_hbm.at[pl.ds(ebase + j * K, K)],
                             emb_v.at[p], se[p])

        def wait_chunk(p):
            pltpu.make_async_copy(x_hbm.at[pl.ds(0, K)], rows_v.at[p], sg[p]).wait()
            pltpu.make_async_copy(emb_hbm.at[pl.ds(0, K)], emb_v.at[p], se[p]).wait()

        start_chunk(0, 0)
        start_chunk(1, 1)

        def body(b, carry):
            for p in range(2):
                j = 2 * b + p
                wait_chunk(p)
                pltpu.sync_copy(emb_v.at[p], acc.at[idx_v.at[p, 1]], add=True)
                pltpu.sync_copy(rows_v.at[p], acc.at[idx_v.at[p, 1]], add=True)

                @pl.when(b < CH // 2 - 1)
                def _():
                    start_chunk(j + 2, p)

            return carry

        lax.fori_loop(0, CH // 2, body, 0)
        plsc.subcore_barrier()

        for t in range(nz):
            off = t * K
            nrows = K if t < nz - 1 else RPT - off
            pltpu.sync_copy(acc.at[pl.ds(r0 + off, nrows)],
                            rows_v.at[0, pl.ds(0, nrows)])
            pltpu.sync_copy(rows_v.at[0, pl.ds(0, nrows)],
                            agg_out.at[cid, pl.ds(r0 + off, nrows)])

    return k


def _tc_body(x_ref, agg_ref, w1_ref, b1_ref,
             g1_ref, bt1_ref, w2_ref, b2_ref, g2_ref, bt2_ref, eps_ref, o_ref):
    agg = agg_ref[0, :N, :] + agg_ref[1, :N, :]
    xx = x_ref[...]
    h = (1.0 + eps_ref[0, 0]) * xx + agg
    z = jnp.dot(h, w1_ref[...], preferred_element_type=jnp.float32) + b1_ref[...]
    m = jnp.mean(z, axis=0, keepdims=True)
    v = jnp.mean((z - m) ** 2, axis=0, keepdims=True)
    z = jnp.maximum((z - m) * lax.rsqrt(v + 1e-5) * g1_ref[...] + bt1_ref[...], 0.0)
    z2 = jnp.dot(z, w2_ref[...], preferred_element_type=jnp.float32) + b2_ref[...]
    m2 = jnp.mean(z2, axis=0, keepdims=True)
    v2 = jnp.mean((z2 - m2) ** 2, axis=0, keepdims=True)
    o_ref[...] = jnp.maximum((z2 - m2) * lax.rsqrt(v2 + 1e-5) * g2_ref[...] + bt2_ref[...], 0.0)


def kernel(x, edge_index, edge_attr, W_e, b_e, W1, b1, g1, beta1, W2, b2, g2, beta2, eps):
    src = edge_index[0].astype(jnp.int32)
    dst = edge_index[1].astype(jnp.int32)
    pad = EPAD - E
    src3d = jnp.concatenate([src, jnp.zeros((pad,), jnp.int32)]).reshape(NT, CH, K)
    dst_pad = N + (jnp.arange(pad, dtype=jnp.int32) % (NP - N))
    dst3d = jnp.concatenate([dst, dst_pad]).reshape(NT, CH, K)
    srcdst = jnp.stack([src3d, dst3d], axis=2)
    ea_pad = jnp.concatenate([edge_attr, jnp.zeros((pad, DE), edge_attr.dtype)])

    emb = _edge_emb(ea_pad, W_e, b_e.reshape(1, DI))
    (agg_pk,) = _sc_aggregate()(x, srcdst, emb)

    out = pl.pallas_call(
        _tc_body,
        out_shape=jax.ShapeDtypeStruct((N, DI), jnp.float32),
    )(x, agg_pk, W1, b1.reshape(1, DH),
      g1.reshape(1, DH), beta1.reshape(1, DH), W2, b2.reshape(1, DI),
      g2.reshape(1, DI), beta2.reshape(1, DI), eps.reshape(1, 1))
    return out


# speedup vs baseline: 1.2770x; 1.0246x over previous
import functools

import jax
import jax.numpy as jnp
from jax import lax
from jax.experimental import pallas as pl
from jax.experimental.pallas import tpu as pltpu
from jax.experimental.pallas import tpu_sc as plsc

N = 10000
NP = 10112
DI = 128
DE = 16
DH = 256
E = 320000
K = 64
CH = 158
EPT = CH * K
NT = 32
EPAD = NT * EPT
RPT = NP // 16
EB = 20224

_MESH = plsc.VectorSubcoreMesh(core_axis_name="c", subcore_axis_name="s")


def _emb_body(ea_ref, we_ref, be_ref, o_ref):
    o_ref[...] = (jnp.dot(ea_ref[...], we_ref[...],
                          preferred_element_type=jnp.float32) + be_ref[...])


def _edge_emb(ea_pad, W_e, b_e2d):
    return pl.pallas_call(
        _emb_body,
        grid=(EPAD // EB,),
        in_specs=[
            pl.BlockSpec((EB, DE), lambda i: (i, 0)),
            pl.BlockSpec((DE, DI), lambda i: (0, 0)),
            pl.BlockSpec((1, DI), lambda i: (0, 0)),
        ],
        out_specs=pl.BlockSpec((EB, DI), lambda i: (i, 0)),
        out_shape=jax.ShapeDtypeStruct((EPAD, DI), jnp.float32),
    )(ea_pad, W_e, b_e2d)


def _sc_aggregate():
    @functools.partial(
        pl.kernel,
        mesh=_MESH,
        out_type=[jax.ShapeDtypeStruct((2, NP, DI), jnp.float32)],
        scratch_types=[
            pltpu.VMEM((2, 2, K), jnp.int32),
            pltpu.VMEM((2, K, DI), jnp.float32),
            pltpu.VMEM((2, K, DI), jnp.float32),
            pltpu.VMEM_SHARED((NP, DI), jnp.float32),
            pltpu.SemaphoreType.DMA,
            pltpu.SemaphoreType.DMA,
            pltpu.SemaphoreType.DMA,
            pltpu.SemaphoreType.DMA,
        ],
    )
    def k(x_hbm, sd_hbm, emb_hbm, agg_out,
          idx_v, rows_v, emb_v, acc, sg0, sg1, se0, se1):
        cid = lax.axis_index("c")
        sid = lax.axis_index("s")
        wid = cid * 16 + sid
        r0 = sid * RPT
        ebase = wid * EPT
        sg = (sg0, sg1)
        se = (se0, se1)

        def zrow(q, c2):
            for c in range(DI // 16):
                rows_v[0, q, pl.ds(c * 16, 16)] = jnp.zeros((16,), jnp.float32)
            return c2

        lax.fori_loop(0, K, zrow, 0)
        nz = RPT // K + 1
        for t in range(nz):
            off = t * K
            nrows = K if t < nz - 1 else RPT - off
            pltpu.sync_copy(rows_v.at[0, pl.ds(0, nrows)],
                            acc.at[pl.ds(r0 + off, nrows)])
        plsc.subcore_barrier()

        def start_chunk(j, p):
            pltpu.sync_copy(sd_hbm.at[wid, j], idx_v.at[p])
            pltpu.async_copy(x_hbm.at[idx_v.at[p, 0]], rows_v.at[p], sg[p])
            pltpu.async_copy(emb_hbm.at[pl.ds(ebase + j * K, K)],
                             emb_v.at[p], se[p])

        def wait_chunk(p):
            pltpu.make_async_copy(x_hbm.at[pl.ds(0, K)], rows_v.at[p], sg[p]).wait()
            pltpu.make_async_copy(emb_hbm.at[pl.ds(0, K)], emb_v.at[p], se[p]).wait()

        start_chunk(0, 0)
        start_chunk(1, 1)

        def body(b, carry):
            for p in range(2):
                j = 2 * b + p
                wait_chunk(p)
                pltpu.sync_copy(emb_v.at[p], acc.at[idx_v.at[p, 1]], add=True)
                pltpu.sync_copy(rows_v.at[p], acc.at[idx_v.at[p, 1]], add=True)

                @pl.when(b < CH // 2 - 1)
                def _():
                    start_chunk(j + 2, p)

            return carry

        lax.fori_loop(0, CH // 2, body, 0)
        plsc.subcore_barrier()

        for t in range(nz):
            off = t * K
            nrows = K if t < nz - 1 else RPT - off
            pltpu.sync_copy(acc.at[pl.ds(r0 + off, nrows)],
                            rows_v.at[0, pl.ds(0, nrows)])
            pltpu.sync_copy(rows_v.at[0, pl.ds(0, nrows)],
                            agg_out.at[cid, pl.ds(r0 + off, nrows)])

    return k


def _tc_body(x_ref, agg_ref, w1_ref, b1_ref,
             g1_ref, bt1_ref, w2_ref, b2_ref, g2_ref, bt2_ref, eps_ref, o_ref):
    agg = agg_ref[0, :N, :] + agg_ref[1, :N, :]
    xx = x_ref[...]
    h = (1.0 + eps_ref[0, 0]) * xx + agg
    z = jnp.dot(h, w1_ref[...], preferred_element_type=jnp.float32) + b1_ref[...]
    m = jnp.mean(z, axis=0, keepdims=True)
    v = jnp.mean((z - m) ** 2, axis=0, keepdims=True)
    z = jnp.maximum((z - m) * lax.rsqrt(v + 1e-5) * g1_ref[...] + bt1_ref[...], 0.0)
    z2 = jnp.dot(z, w2_ref[...], preferred_element_type=jnp.float32) + b2_ref[...]
    m2 = jnp.mean(z2, axis=0, keepdims=True)
    v2 = jnp.mean((z2 - m2) ** 2, axis=0, keepdims=True)
    o_ref[...] = jnp.maximum((z2 - m2) * lax.rsqrt(v2 + 1e-5) * g2_ref[...] + bt2_ref[...], 0.0)


def kernel(x, edge_index, edge_attr, W_e, b_e, W1, b1, g1, beta1, W2, b2, g2, beta2, eps):
    src = edge_index[0].astype(jnp.int32)
    dst = edge_index[1].astype(jnp.int32)
    pad = EPAD - E
    src3d = jnp.concatenate([src, jnp.zeros((pad,), jnp.int32)]).reshape(NT, CH, K)
    dst_pad = N + (jnp.arange(pad, dtype=jnp.int32) % (NP - N))
    dst3d = jnp.concatenate([dst, dst_pad]).reshape(NT, CH, K)
    srcdst = jnp.stack([src3d, dst3d], axis=2)
    ea_pad = jnp.concatenate([edge_attr, jnp.zeros((pad, DE), edge_attr.dtype)])

    emb = _edge_emb(ea_pad, W_e, b_e.reshape(1, DI))
    (agg_pk,) = _sc_aggregate()(x, srcdst, emb)

    out = pl.pallas_call(
        _tc_body,
        out_shape=jax.ShapeDtypeStruct((N, DI), jnp.float32),
    )(x, agg_pk, W1, b1.reshape(1, DH),
      g1.reshape(1, DH), beta1.reshape(1, DH), W2, b2.reshape(1, DI),
      g2.reshape(1, DI), beta2.reshape(1, DI), eps.reshape(1, 1))
    return out
